# baseline (speedup 1.0000x reference)
import functools

import numpy as np
import jax
import jax.numpy as jnp
from jax import lax
from jax.experimental import pallas as pl
from jax.experimental.pallas import tpu as pltpu
from jax.experimental.pallas import tpu_sc as plsc

_NC = 2
_NS = 16
_NW = _NC * _NS

_B = 16384
_V = 1000000
_D = 32
_FULL_TILES = _V // 128
_TAIL_BASE = _FULL_TILES * 128
_TPW = _FULL_TILES // _NW
_TREM = _FULL_TILES % _NW
_CT = 4
_CHU = _CT * 128
_NCHUNK = -(-(_TPW + 1) // _CT)
_OUTR = _B + 16
_SENT = 2 ** 30


def _const_tables():
    t0 = np.array([w * _TPW + min(w, _TREM) for w in range(_NW)], np.int64)
    t1 = t0 + _TPW + (np.arange(_NW) < _TREM)
    bounds = np.concatenate([np.repeat(t0, 16), np.repeat(t1, 16)])
    karr = np.repeat(np.arange(_NCHUNK, dtype=np.int64), 16)
    return bounds.astype(np.int32), karr.astype(np.int32)


@functools.lru_cache(maxsize=None)
def _make_scan():
    mesh = plsc.VectorSubcoreMesh(core_axis_name="c", subcore_axis_name="s")

    @functools.partial(
        pl.kernel,
        out_type=jax.ShapeDtypeStruct((_OUTR, 128), jnp.float32),
        mesh=mesh,
        scratch_types=[
            pltpu.VMEM((_B,), jnp.int32),
            pltpu.VMEM((_B + 32,), jnp.int32),
            pltpu.VMEM((_B + 32,), jnp.int32),
            pltpu.VMEM((_B + 32,), jnp.int32),
            pltpu.VMEM((_B + 32,), jnp.int32),
            pltpu.VMEM((2 * _D, _CHU), jnp.float32),
            pltpu.VMEM((16, 128), jnp.float32),
            pltpu.VMEM((16, 128), jnp.float32),
            pltpu.VMEM((16,), jnp.int32),
            pltpu.VMEM((16,), jnp.int32),
            pltpu.VMEM((2 * 16,), jnp.int32),
            pltpu.VMEM((_NCHUNK * 16,), jnp.int32),
            pltpu.VMEM((16,), jnp.int32),
            pltpu.VMEM((16,), jnp.int32),
            pltpu.SMEM((4,), jnp.int32),
            pltpu.SemaphoreType.DMA,
            pltpu.SemaphoreType.DMA,
        ],
        compiler_params=pltpu.CompilerParams(needs_layout_passes=False),
    )
    def scan(tablT, seg_hbm, bounds_hbm, karr_hbm, out_hbm,
             seg_v, hit_r, hit_p, c_c, c_p, slab, st0, st1, pos0, pos1,
             bnd_v, kv_v, nbuf, mbuf, scnt, sem, sems):
        wid = lax.axis_index("s") * _NC + lax.axis_index("c")
        t0 = wid * _TPW + jnp.minimum(wid, _TREM)
        t1 = t0 + _TPW + jnp.where(wid < _TREM, 1, 0)

        pltpu.sync_copy(seg_hbm, seg_v)
        pltpu.sync_copy(bounds_hbm.at[pl.ds(wid * 16, 16)],
                        bnd_v.at[pl.ds(0, 16)])
        pltpu.sync_copy(bounds_hbm.at[pl.ds(_NW * 16 + wid * 16, 16)],
                        bnd_v.at[pl.ds(16, 16)])
        pltpu.sync_copy(karr_hbm, kv_v)

        lanes = lax.iota(jnp.int32, 16)
        t0_v = bnd_v[pl.ds(0, 16)]
        t1_v = bnd_v[pl.ds(16, 16)]
        nbuf[pl.ds(0, 16)] = lanes * 0

        def phase_a(i, posv):
            r = seg_v[pl.ds(i * 16, 16)]
            tile = lax.shift_right_logical(r, 7)
            msk = (tile >= t0_v) & (tile < t1_v)
            m32 = jnp.where(msk, 1, 0)
            nv = nbuf[pl.ds(0, 16)]
            offs = jnp.where(msk, nv + jnp.cumsum(m32) - 1, _B + 16)
            plsc.store_scatter(hit_r, [offs], r)
            plsc.store_scatter(hit_p, [offs], posv)
            nbuf[pl.ds(0, 16)] = nv + plsc.all_reduce_population_count(msk)
            return posv + 16

        lax.fori_loop(0, _B // 16, phase_a, lanes)
        scnt[0] = jnp.sum(nbuf[pl.ds(0, 16)], axis=0) // 16
        n_s = scnt[0]
        hit_r[pl.ds(n_s, 16)] = lanes * 0 + _SENT
        hit_p[pl.ds(n_s, 16)] = lanes * 0 + _B
        nvec = (n_s + 15) // 16

        def slab_rows(par):
            return slab.at[pl.ds(par * _D, _D), :]

        def chunk_base(k):
            lo = t0 + k * _CT
            cbase = jnp.minimum(lo, t1 - _CT) * 128
            return pl.multiple_of(cbase, 128)

        pltpu.async_copy(
            tablT.at[:, pl.ds(chunk_base(jnp.int32(0)), _CHU)],
            slab_rows(0), sems)

        def chunk_body(k, bc):
            par = lax.rem(k, jnp.int32(2))
            pltpu.make_async_copy(
                tablT.at[:, pl.ds(0, _CHU)], slab_rows(0), sems).wait()

            @pl.when(k < _NCHUNK - 1)
            def _prefetch():
                nxt = lax.rem(k + 1, jnp.int32(2))
                pltpu.async_copy(
                    tablT.at[:, pl.ds(chunk_base(k + 1), _CHU)],
                    slab_rows(nxt), sems)

            kv = kv_v[pl.ds(k * 16, 16)]
            lo_v = t0_v + kv * _CT
            hi_v = jnp.minimum(lo_v + _CT, t1_v)
            cb_v = jnp.minimum(lo_v, t1_v - _CT) * 128
            par_v = lax.rem(kv, 2) * _D
            mbuf[pl.ds(0, 16)] = lanes * 0

            def scanhits(j, carry):
                rj = hit_r[pl.ds(j * 16, 16)]
                pj = hit_p[pl.ds(j * 16, 16)]
                tile = lax.shift_right_logical(rj, 7)
                msk = (tile >= lo_v) & (tile < hi_v)
                m32 = jnp.where(msk, 1, 0)
                mv = mbuf[pl.ds(0, 16)]
                offs = jnp.where(msk, mv + jnp.cumsum(m32) - 1, _B + 16)
                plsc.store_scatter(c_c, [offs], rj - cb_v)
                plsc.store_scatter(c_p, [offs], pj)
                mbuf[pl.ds(0, 16)] = mv + plsc.all_reduce_population_count(msk)
                return carry

            lax.fori_loop(0, nvec, scanhits, jnp.int32(0))
            scnt[1] = jnp.sum(mbuf[pl.ds(0, 16)], axis=0) // 16
            m_s = scnt[1]
            c_c[pl.ds(m_s, 16)] = lanes * 0
            c_p[pl.ds(m_s, 16)] = lanes * 0 + _B

            def batch_body(b, bc):
                cvec = c_c[pl.ds(b * 16, 16)]
                pvec = c_p[pl.ds(b * 16, 16)]
                slot = lax.rem(bc, jnp.int32(2))

                @pl.when(bc >= 2)
                def _drain():
                    pltpu.make_async_copy(
                        seg_hbm.at[pl.ds(0, 2048)],
                        seg_v.at[pl.ds(0, 2048)],
                        sem,
                    ).wait()

                vals = [
                    plsc.load_gather(slab, [par_v + d, cvec])
                    for d in range(_D)
                ]

                @pl.when(slot == 0)
                def _s0():
                    pos0[pl.ds(0, 16)] = pvec
                    for d in range(_D):
                        plsc.store_scatter(
                            st0, [lanes, jnp.full((16,), d, jnp.int32)],
                            vals[d])
                    pltpu.async_copy(st0, out_hbm.at[pos0], sem)

                @pl.when(slot == 1)
                def _s1():
                    pos1[pl.ds(0, 16)] = pvec
                    for d in range(_D):
                        plsc.store_scatter(
                            st1, [lanes, jnp.full((16,), d, jnp.int32)],
                            vals[d])
                    pltpu.async_copy(st1, out_hbm.at[pos1], sem)

                return bc + 1

            return lax.fori_loop(0, (m_s + 15) // 16, batch_body, bc)

        bc = lax.fori_loop(0, _NCHUNK, chunk_body, jnp.int32(0))

        def drain(i, carry):
            @pl.when(i < jnp.minimum(bc, 2))
            def _w():
                pltpu.make_async_copy(
                    seg_hbm.at[pl.ds(0, 2048)],
                    seg_v.at[pl.ds(0, 2048)],
                    sem,
                ).wait()
            return carry

        lax.fori_loop(0, 2, drain, jnp.int32(0))

    return scan


def _mlp_body(emb128, seg, t, c, tail, w1e, w1t, w1c, b1, w2, b2, out):
    blk = out.shape[0]
    s = seg[...]
    emb = emb128[...][:, :_D]
    oh = (lax.broadcasted_iota(jnp.int32, (blk, 64), 1)
          == (s - _TAIL_BASE)).astype(jnp.float32)
    embt = jnp.dot(oh, tail[...], preferred_element_type=jnp.float32)
    emb = jnp.where(s >= _TAIL_BASE, embt, emb)
    h = jnp.dot(emb, w1e[...], preferred_element_type=jnp.float32)
    h = h + jnp.dot(t[...], w1t[...], preferred_element_type=jnp.float32)
    h = h + jnp.dot(c[...], w1c[...], preferred_element_type=jnp.float32)
    h = jnp.maximum(h + b1[...], 0.0)
    out[...] = jnp.sum(h * w2[...], axis=1, keepdims=True) + b2[...]


@functools.lru_cache(maxsize=None)
def _make_mlp(B, D, T, C, H, blk):
    grid = B // blk
    full = lambda shape: pl.BlockSpec(shape, lambda i: (0, 0))
    rows = lambda w: pl.BlockSpec((blk, w), lambda i: (i, 0))
    return pl.pallas_call(
        _mlp_body,
        grid=(grid,),
        in_specs=[
            rows(128), rows(1), rows(T), rows(C),
            full((64, D)),
            full((D, H)), full((T, H)), full((C, H)),
            full((1, H)), full((1, H)), full((1, 1)),
        ],
        out_specs=rows(1),
        out_shape=jax.ShapeDtypeStruct((B, 1), jnp.float32),
    )


def kernel(seg_idx, temporal, context_flags, table, W1, b1, W2, b2):
    B = seg_idx.shape[0]
    V, D = table.shape
    T = temporal.shape[1]
    C = context_flags.shape[1]
    H = W1.shape[1]

    idx = seg_idx.astype(jnp.int32)
    bounds, karr = _const_tables()
    emb128 = _make_scan()(table.T, idx, jnp.asarray(bounds),
                          jnp.asarray(karr))

    tail = jnp.pad(table[_TAIL_BASE:], ((0, 64 - (V - _TAIL_BASE)), (0, 0)))
    out = _make_mlp(B, D, T, C, H, 2048)(
        emb128, idx.reshape(B, 1), temporal, context_flags, tail,
        W1[:D], W1[D:D + T], W1[D + T:],
        b1.reshape(1, H), W2.reshape(1, H), b2.reshape(1, 1),
    )
    return out


# speedup vs baseline: 1.5016x; 1.5016x over previous
import functools

import jax
import jax.numpy as jnp
from jax import lax
from jax.experimental import pallas as pl
from jax.experimental.pallas import tpu as pltpu
from jax.experimental.pallas import tpu_sc as plsc

_NC = 2
_NS = 16
_NW = _NC * _NS
_CHUNK = 128


@functools.lru_cache(maxsize=None)
def _make_gather(V, D, B):
    b_per_w = B // _NW
    n_chunks = b_per_w // _CHUNK
    mesh = plsc.VectorSubcoreMesh(core_axis_name="c", subcore_axis_name="s")

    @functools.partial(
        pl.kernel,
        out_type=jax.ShapeDtypeStruct((B, D), jnp.float32),
        mesh=mesh,
        scratch_types=[
            pltpu.VMEM((n_chunks, _CHUNK), jnp.int32),
            pltpu.VMEM((b_per_w, D), jnp.float32),
            pltpu.SemaphoreType.DMA,
        ],
        compiler_params=pltpu.CompilerParams(use_tc_tiling_on_sc=False),
    )
    def gather(table_hbm, idx_hbm, out_hbm, idx_v, rows_v, sem):
        wid = lax.axis_index("s") * _NC + lax.axis_index("c")
        pltpu.sync_copy(idx_hbm.at[pl.ds(wid * n_chunks, n_chunks)], idx_v)
        copies = [
            pltpu.async_copy(
                table_hbm.at[idx_v.at[j]],
                rows_v.at[pl.ds(j * _CHUNK, _CHUNK)],
                sem,
            )
            for j in range(n_chunks)
        ]
        for c in copies:
            c.wait()
        pltpu.sync_copy(rows_v, out_hbm.at[pl.ds(wid * b_per_w, b_per_w)])

    return gather


def _mlp_body(emb, t, c, w1e, w1t, w1c, b1, w2, b2, out):
    h = jnp.dot(emb[...], w1e[...], preferred_element_type=jnp.float32)
    h = h + jnp.dot(t[...], w1t[...], preferred_element_type=jnp.float32)
    h = h + jnp.dot(c[...], w1c[...], preferred_element_type=jnp.float32)
    h = jnp.maximum(h + b1[...], 0.0)
    out[...] = jnp.sum(h * w2[...], axis=1, keepdims=True) + b2[...]


@functools.lru_cache(maxsize=None)
def _make_mlp(B, D, T, C, H, blk):
    grid = B // blk
    full = lambda shape: pl.BlockSpec(shape, lambda i: (0, 0))
    rows = lambda w: pl.BlockSpec((blk, w), lambda i: (i, 0))
    return pl.pallas_call(
        _mlp_body,
        grid=(grid,),
        in_specs=[
            rows(D), rows(T), rows(C),
            full((D, H)), full((T, H)), full((C, H)),
            full((1, H)), full((1, H)), full((1, 1)),
        ],
        out_specs=rows(1),
        out_shape=jax.ShapeDtypeStruct((B, 1), jnp.float32),
    )


def kernel(seg_idx, temporal, context_flags, table, W1, b1, W2, b2):
    B = seg_idx.shape[0]
    V, D = table.shape
    T = temporal.shape[1]
    C = context_flags.shape[1]
    H = W1.shape[1]

    idx = seg_idx.astype(jnp.int32).reshape(B // _CHUNK, _CHUNK)
    emb = _make_gather(V, D, B)(table, idx)

    out = _make_mlp(B, D, T, C, H, 2048)(
        emb, temporal, context_flags,
        W1[:D], W1[D:D + T], W1[D + T:],
        b1.reshape(1, H), W2.reshape(1, H), b2.reshape(1, 1),
    )
    return out
